# Initial kernel scaffold; baseline (speedup 1.0000x reference)
import jax
import jax.numpy as jnp
from jax.experimental import pallas as pl


def kernel(pos, g_pmi, g_emb, node_features, W1_pmi, W2_pmi, W1_emb, W2_emb, Wp1, bp1, Wp2, bp2):
    raise NotImplementedError("write your pallas kernel here")


# speedup vs baseline: 4.3551x; 4.3551x over previous
import functools

import jax
import jax.numpy as jnp
from jax import lax
from jax.experimental import pallas as pl
from jax.experimental.pallas import tpu as pltpu
from jax.experimental.pallas import tpu_sc as plsc

N = 10000
E = 160000
N_INP = 256
N_HID = 512
N_OUT = 128
TAU = 0.8
LAM = 0.5

_B = 128
_NBATCH = E // _B
_NSUB = 16
_WB = 632
_WB_LAST = N - (_NSUB - 1) * _WB


def _sc_mesh():
    return plsc.VectorSubcoreMesh(
        core_axis_name="c", subcore_axis_name="s", num_cores=2, num_subcores=_NSUB
    )


def _tile_batches(s):
    base = _NBATCH // _NSUB
    rem = _NBATCH - base * _NSUB
    nb = jnp.where(s < rem, base + 1, base)
    start = s * base + jnp.minimum(s, rem)
    return start, nb


def _writeback(s, copy_one):
    r0 = pl.multiple_of(s * _WB, 8)

    @pl.when(s < _NSUB - 1)
    def _():
        copy_one(r0, _WB)

    @pl.when(s == _NSUB - 1)
    def _():
        copy_one((_NSUB - 1) * _WB, _WB_LAST)


def _sc_degrees(g_pmi, g_emb):
    ones = jnp.ones((_B, 128), jnp.float32)
    zdeg = jnp.zeros((N, 128), jnp.float32)

    @functools.partial(
        pl.kernel,
        out_type=jax.ShapeDtypeStruct((2, 2, N, 128), jnp.float32),
        mesh=_sc_mesh(),
        scratch_types=[
            pltpu.VMEM((_B,), jnp.int32),
            pltpu.VMEM((_B, 128), jnp.float32),
            pltpu.VMEM_SHARED((N, 128), jnp.float32),
        ],
    )
    def k(gp_h, ge_h, z_h, ones_h, out_h, sidx, ones_v, acc):
        c = lax.axis_index("c")
        s = lax.axis_index("s")
        pltpu.sync_copy(ones_h, ones_v)
        start, nb = _tile_batches(s)

        for kind in (0, 1):
            @pl.when(s == 0)
            def _():
                pltpu.sync_copy(z_h, acc)

            plsc.subcore_barrier()

            def run(g_h):
                def body(i, carry):
                    off = (start + i) * _B
                    pltpu.sync_copy(g_h.at[kind, pl.ds(off, _B)], sidx)
                    pltpu.sync_copy(ones_v, acc.at[sidx], add=True)
                    return carry

                lax.fori_loop(0, nb, body, 0)

            @pl.when(c == 0)
            def _():
                run(gp_h)

            @pl.when(c == 1)
            def _():
                run(ge_h)

            plsc.subcore_barrier()

            def copy_one(r0, nr):
                pltpu.sync_copy(acc.at[pl.ds(r0, nr)], out_h.at[c, kind, pl.ds(r0, nr)])

            _writeback(s, copy_one)
            plsc.subcore_barrier()

    return k(g_pmi, g_emb, zdeg, ones)


def _sc_scatter(v_pmi, v_emb, g_pmi, g_emb):
    z128 = jnp.zeros((N, 128), jnp.float32)

    @functools.partial(
        pl.kernel,
        out_type=jax.ShapeDtypeStruct((2, N, 128), jnp.float32),
        mesh=_sc_mesh(),
        scratch_types=[
            pltpu.VMEM((_B,), jnp.int32),
            pltpu.VMEM((_B,), jnp.int32),
            pltpu.VMEM((_B, 128), jnp.float32),
            pltpu.VMEM_SHARED((N, 128), jnp.float32),
            pltpu.SemaphoreType.DMA,
        ],
    )
    def k(vp_h, ve_h, gp_h, ge_h, z_h, out_h, sidx, didx, rows, acc, sem):
        c = lax.axis_index("c")
        s = lax.axis_index("s")

        @pl.when(s == 0)
        def _():
            pltpu.sync_copy(z_h, acc)

        plsc.subcore_barrier()
        start, nb = _tile_batches(s)

        def run(v_h, g_h):
            def body(i, carry):
                off = (start + i) * _B
                pltpu.sync_copy(g_h.at[0, pl.ds(off, _B)], sidx)
                pltpu.sync_copy(g_h.at[1, pl.ds(off, _B)], didx)
                pltpu.async_copy(v_h.at[sidx], rows, sem).wait()
                pltpu.sync_copy(rows, acc.at[didx], add=True)
                return carry

            lax.fori_loop(0, nb, body, 0)

        @pl.when(c == 0)
        def _():
            run(vp_h, gp_h)

        @pl.when(c == 1)
        def _():
            run(ve_h, ge_h)

        plsc.subcore_barrier()

        def copy_one(r0, nr):
            pltpu.sync_copy(acc.at[pl.ds(r0, nr)], out_h.at[c, pl.ds(r0, nr)])

        _writeback(s, copy_one)

    return k(v_pmi, v_emb, g_pmi, g_emb, z128)


def _inv_sqrt_deg(d16):
    d0 = d16[:, 0:1]
    return jnp.where(d0 > 0, lax.rsqrt(jnp.where(d0 > 0, d0, 1.0)), 0.0)


def _prep(x, deg_out_pmi, deg_out_emb):
    T = 2000

    def body(x_r, dp_r, de_r, xp_r, xe_r):
        xv = x_r[...]
        xp_r[...] = xv * _inv_sqrt_deg(dp_r[...])
        xe_r[...] = xv * _inv_sqrt_deg(de_r[...])

    return pl.pallas_call(
        body,
        grid=(N // T,),
        in_specs=[
            pl.BlockSpec((T, N_INP), lambda i: (i, 0)),
            pl.BlockSpec((T, 128), lambda i: (i, 0)),
            pl.BlockSpec((T, 128), lambda i: (i, 0)),
        ],
        out_specs=[pl.BlockSpec((T, N_INP), lambda i: (i, 0))] * 2,
        out_shape=[jax.ShapeDtypeStruct((N, N_INP), jnp.float32)] * 2,
    )(x, deg_out_pmi, deg_out_emb)


def _layer(a0, a1, deg_in, deg_out, W1, W2):
    T = 2000

    def body(a0_r, a1_r, di_r, do_r, w1_r, w2_r, y_r):
        ndst = _inv_sqrt_deg(di_r[...])
        nsrc = _inv_sqrt_deg(do_r[...])
        w1 = w1_r[...]
        h = jnp.dot(a0_r[...] * ndst, w1[0:128, :], preferred_element_type=jnp.float32)
        h += jnp.dot(a1_r[...] * ndst, w1[128:256, :], preferred_element_type=jnp.float32)
        h = jnp.maximum(h, 0.0) * nsrc
        y_r[...] = jnp.dot(h, w2_r[...], preferred_element_type=jnp.float32)

    return pl.pallas_call(
        body,
        grid=(N // T,),
        in_specs=[
            pl.BlockSpec((T, 128), lambda i: (i, 0)),
            pl.BlockSpec((T, 128), lambda i: (i, 0)),
            pl.BlockSpec((T, 128), lambda i: (i, 0)),
            pl.BlockSpec((T, 128), lambda i: (i, 0)),
            pl.BlockSpec((N_INP, N_HID), lambda i: (0, 0)),
            pl.BlockSpec((N_HID, N_OUT), lambda i: (0, 0)),
        ],
        out_specs=pl.BlockSpec((T, N_OUT), lambda i: (i, 0)),
        out_shape=jax.ShapeDtypeStruct((N, N_OUT), jnp.float32),
    )(a0, a1, deg_in, deg_out, W1, W2)


def _zproj(q, deg_in, Wp1, bp1, Wp2, bp2):
    T = 2000

    def body(q_r, di_r, w1_r, b1_r, w2_r, b2_r, z_r, n_r):
        z = q_r[...] * _inv_sqrt_deg(di_r[...])
        z_r[...] = z
        u = jnp.dot(z, w1_r[...], preferred_element_type=jnp.float32) + b1_r[...]
        e = jnp.where(u > 0, u, jnp.exp(jnp.minimum(u, 0.0)) - 1.0)
        p = jnp.dot(e, w2_r[...], preferred_element_type=jnp.float32) + b2_r[...]
        nrm = jnp.sqrt(jnp.sum(p * p, axis=1, keepdims=True))
        n_r[...] = p / (nrm + 1e-8)

    return pl.pallas_call(
        body,
        grid=(N // T,),
        in_specs=[
            pl.BlockSpec((T, N_OUT), lambda i: (i, 0)),
            pl.BlockSpec((T, 128), lambda i: (i, 0)),
            pl.BlockSpec((N_OUT, N_OUT), lambda i: (0, 0)),
            pl.BlockSpec((1, N_OUT), lambda i: (0, 0)),
            pl.BlockSpec((N_OUT, N_OUT), lambda i: (0, 0)),
            pl.BlockSpec((1, N_OUT), lambda i: (0, 0)),
        ],
        out_specs=[pl.BlockSpec((T, N_OUT), lambda i: (i, 0))] * 2,
        out_shape=[jax.ShapeDtypeStruct((N, N_OUT), jnp.float32)] * 2,
    )(q, deg_in, Wp1, bp1.reshape(1, N_OUT), Wp2, bp2.reshape(1, N_OUT))


def _contrast_half(na, nb, pos):
    T = 200

    def body(na_r, nb_r, pos_r, out_r, acc):
        i = pl.program_id(0)
        sim = jnp.exp(
            lax.dot_general(
                na_r[...], nb_r[...], (((1,), (1,)), ((), ())),
                preferred_element_type=jnp.float32,
            )
            * (1.0 / TAU)
        )
        rs = jnp.sum(sim, axis=1)
        sp = jnp.sum(sim * pos_r[...], axis=1)
        term = jnp.log(sp / (rs + 1e-8) + 1e-8)
        ssum = jnp.sum(term)

        @pl.when(i == 0)
        def _():
            acc[0] = 0.0

        acc[0] += ssum

        @pl.when(i == pl.num_programs(0) - 1)
        def _():
            out_r[...] = jnp.full((1, 128), acc[0], jnp.float32)

    return pl.pallas_call(
        body,
        grid=(N // T,),
        in_specs=[
            pl.BlockSpec((T, N_OUT), lambda i: (i, 0)),
            pl.BlockSpec((N, N_OUT), lambda i: (0, 0)),
            pl.BlockSpec((T, N), lambda i: (i, 0)),
        ],
        out_specs=pl.BlockSpec((1, 128), lambda i: (0, 0)),
        out_shape=jax.ShapeDtypeStruct((1, 128), jnp.float32),
        scratch_shapes=[pltpu.SMEM((1,), jnp.float32)],
    )(na, nb, pos)


def kernel(pos, g_pmi, g_emb, node_features, W1_pmi, W2_pmi, W1_emb, W2_emb, Wp1, bp1, Wp2, bp2):
    deg = _sc_degrees(g_pmi, g_emb)
    dop, dip = deg[0, 0], deg[0, 1]
    doe, die = deg[1, 0], deg[1, 1]

    xs_p, xs_e = _prep(node_features, dop, doe)
    a0 = _sc_scatter(xs_p[:, :128], xs_e[:, :128], g_pmi, g_emb)
    a1 = _sc_scatter(xs_p[:, 128:], xs_e[:, 128:], g_pmi, g_emb)

    y_p = _layer(a0[0], a1[0], dip, dop, W1_pmi, W2_pmi)
    y_e = _layer(a0[1], a1[1], die, doe, W1_emb, W2_emb)
    q = _sc_scatter(y_p, y_e, g_pmi, g_emb)

    z_p, n1 = _zproj(q[0], dip, Wp1, bp1, Wp2, bp2)
    z_e, n2 = _zproj(q[1], die, Wp1, bp1, Wp2, bp2)

    lA = _contrast_half(n1, n2, pos)[0, 0]
    lB = _contrast_half(n2, n1, pos)[0, 0]
    loss = LAM * (-lA / N) + (1.0 - LAM) * (-lB / N)
    return (z_p, z_e, loss)


# speedup vs baseline: 6.5394x; 1.5016x over previous
import functools

import jax
import jax.numpy as jnp
from jax import lax
from jax.experimental import pallas as pl
from jax.experimental.pallas import tpu as pltpu
from jax.experimental.pallas import tpu_sc as plsc

N = 10000
E = 160000
N_INP = 256
N_HID = 512
N_OUT = 128
TAU = 0.8
LAM = 0.5

_B = 128
_NBATCH = E // _B
_NSUB = 16
_MAXB = 80
_HC = 40
_GPAD = _NSUB * _MAXB
_WB = 632
_WB_LAST = N - (_NSUB - 1) * _WB


def _edges_2d(g):
    return jnp.pad(g.reshape(2, _NBATCH, _B), ((0, 0), (0, _GPAD - _NBATCH), (0, 0)))


def _sc_mesh():
    return plsc.VectorSubcoreMesh(
        core_axis_name="c", subcore_axis_name="s", num_cores=2, num_subcores=_NSUB
    )


def _tile_batches(s):
    start = pl.multiple_of(s * _MAXB, 8)
    nb = jnp.minimum(_MAXB, _NBATCH - s * _MAXB)
    return start, nb


def _writeback(s, copy_one):
    r0 = pl.multiple_of(s * _WB, 8)

    @pl.when(s < _NSUB - 1)
    def _():
        copy_one(r0, _WB)

    @pl.when(s == _NSUB - 1)
    def _():
        copy_one((_NSUB - 1) * _WB, _WB_LAST)


def _sc_degrees(g_pmi, g_emb):
    ones = jnp.ones((_B, 128), jnp.float32)
    zdeg = jnp.zeros((N, 128), jnp.float32)

    @functools.partial(
        pl.kernel,
        out_type=jax.ShapeDtypeStruct((2, 2, N, 128), jnp.float32),
        mesh=_sc_mesh(),
        scratch_types=[
            pltpu.VMEM((_MAXB, _B), jnp.int32),
            pltpu.VMEM((_B, 128), jnp.float32),
            pltpu.VMEM_SHARED((N, 128), jnp.float32),
        ],
    )
    def k(gp_h, ge_h, z_h, ones_h, out_h, sidx, ones_v, acc):
        c = lax.axis_index("c")
        s = lax.axis_index("s")
        pltpu.sync_copy(ones_h, ones_v)
        start, nb = _tile_batches(s)

        for kind in (0, 1):
            @pl.when(c == 0)
            def _():
                pltpu.sync_copy(gp_h.at[kind, pl.ds(start, _MAXB)], sidx)

            @pl.when(c == 1)
            def _():
                pltpu.sync_copy(ge_h.at[kind, pl.ds(start, _MAXB)], sidx)

            @pl.when(s == 0)
            def _():
                pltpu.sync_copy(z_h, acc)

            plsc.subcore_barrier()

            def body(i, carry):
                pltpu.sync_copy(ones_v, acc.at[sidx.at[i]], add=True)
                return carry

            lax.fori_loop(0, nb, body, 0)

            plsc.subcore_barrier()

            def copy_one(r0, nr):
                pltpu.sync_copy(acc.at[pl.ds(r0, nr)], out_h.at[c, kind, pl.ds(r0, nr)])

            _writeback(s, copy_one)
            plsc.subcore_barrier()

    return k(_edges_2d(g_pmi), _edges_2d(g_emb), zdeg, ones)


def _sc_scatter(v_pmi, v_emb, g_pmi, g_emb):
    z128 = jnp.zeros((N, 128), jnp.float32)

    @functools.partial(
        pl.kernel,
        out_type=jax.ShapeDtypeStruct((2, N, 128), jnp.float32),
        mesh=_sc_mesh(),
        scratch_types=[
            pltpu.VMEM((_HC, _B), jnp.int32),
            pltpu.VMEM((_HC, _B), jnp.int32),
            pltpu.VMEM((_B, 128), jnp.float32),
            pltpu.VMEM((_B, 128), jnp.float32),
            pltpu.VMEM_SHARED((N, 128), jnp.float32),
            pltpu.SemaphoreType.DMA,
            pltpu.SemaphoreType.DMA,
        ],
    )
    def k(vp_h, ve_h, gp_h, ge_h, z_h, out_h, sidx, didx, rows0, rows1, acc, sem0, sem1):
        c = lax.axis_index("c")
        s = lax.axis_index("s")
        start, nb = _tile_batches(s)

        @pl.when(s == 0)
        def _():
            pltpu.sync_copy(z_h, acc)

        plsc.subcore_barrier()

        def run(v_h, g_h):
            for h in range(_MAXB // _HC):
                h0 = pl.multiple_of(start + h * _HC, 8)
                nh = jnp.clip(nb - h * _HC, 0, _HC)

                @pl.when(nh > 0)
                def _():
                    pltpu.sync_copy(g_h.at[0, pl.ds(h0, _HC)], sidx)
                    pltpu.sync_copy(g_h.at[1, pl.ds(h0, _HC)], didx)
                    pltpu.async_copy(v_h.at[sidx.at[0]], rows0, sem0)

                    def body2(j, carry):
                        i0 = 2 * j
                        i1 = i0 + 1
                        pltpu.async_copy(v_h.at[sidx.at[i1]], rows1, sem1)
                        pltpu.make_async_copy(v_h.at[sidx.at[i0]], rows0, sem0).wait()
                        pltpu.sync_copy(rows0, acc.at[didx.at[i0]], add=True)

                        @pl.when(i0 + 2 < nh)
                        def _():
                            pltpu.async_copy(v_h.at[sidx.at[i0 + 2]], rows0, sem0)

                        pltpu.make_async_copy(v_h.at[sidx.at[i1]], rows1, sem1).wait()
                        pltpu.sync_copy(rows1, acc.at[didx.at[i1]], add=True)
                        return carry

                    lax.fori_loop(0, nh // 2, body2, 0)

        @pl.when(c == 0)
        def _():
            run(vp_h, gp_h)

        @pl.when(c == 1)
        def _():
            run(ve_h, ge_h)

        plsc.subcore_barrier()

        def copy_one(r0, nr):
            pltpu.sync_copy(acc.at[pl.ds(r0, nr)], out_h.at[c, pl.ds(r0, nr)])

        _writeback(s, copy_one)

    return k(v_pmi, v_emb, _edges_2d(g_pmi), _edges_2d(g_emb), z128)


def _inv_sqrt_deg(d16):
    d0 = d16[:, 0:1]
    return jnp.where(d0 > 0, lax.rsqrt(jnp.where(d0 > 0, d0, 1.0)), 0.0)


def _prep(x, deg_out_pmi, deg_out_emb):
    T = 2000

    def body(x_r, dp_r, de_r, xp_r, xe_r):
        xv = x_r[...]
        xp_r[...] = xv * _inv_sqrt_deg(dp_r[...])
        xe_r[...] = xv * _inv_sqrt_deg(de_r[...])

    return pl.pallas_call(
        body,
        grid=(N // T,),
        in_specs=[
            pl.BlockSpec((T, N_INP), lambda i: (i, 0)),
            pl.BlockSpec((T, 128), lambda i: (i, 0)),
            pl.BlockSpec((T, 128), lambda i: (i, 0)),
        ],
        out_specs=[pl.BlockSpec((T, N_INP), lambda i: (i, 0))] * 2,
        out_shape=[jax.ShapeDtypeStruct((N, N_INP), jnp.float32)] * 2,
    )(x, deg_out_pmi, deg_out_emb)


def _layer(a0, a1, deg_in, deg_out, W1, W2):
    T = 2000

    def body(a0_r, a1_r, di_r, do_r, w1_r, w2_r, y_r):
        ndst = _inv_sqrt_deg(di_r[...])
        nsrc = _inv_sqrt_deg(do_r[...])
        w1 = w1_r[...]
        h = jnp.dot(a0_r[...] * ndst, w1[0:128, :], preferred_element_type=jnp.float32)
        h += jnp.dot(a1_r[...] * ndst, w1[128:256, :], preferred_element_type=jnp.float32)
        h = jnp.maximum(h, 0.0) * nsrc
        y_r[...] = jnp.dot(h, w2_r[...], preferred_element_type=jnp.float32)

    return pl.pallas_call(
        body,
        grid=(N // T,),
        in_specs=[
            pl.BlockSpec((T, 128), lambda i: (i, 0)),
            pl.BlockSpec((T, 128), lambda i: (i, 0)),
            pl.BlockSpec((T, 128), lambda i: (i, 0)),
            pl.BlockSpec((T, 128), lambda i: (i, 0)),
            pl.BlockSpec((N_INP, N_HID), lambda i: (0, 0)),
            pl.BlockSpec((N_HID, N_OUT), lambda i: (0, 0)),
        ],
        out_specs=pl.BlockSpec((T, N_OUT), lambda i: (i, 0)),
        out_shape=jax.ShapeDtypeStruct((N, N_OUT), jnp.float32),
    )(a0, a1, deg_in, deg_out, W1, W2)


def _zproj(q, deg_in, Wp1, bp1, Wp2, bp2):
    T = 2000

    def body(q_r, di_r, w1_r, b1_r, w2_r, b2_r, z_r, n_r):
        z = q_r[...] * _inv_sqrt_deg(di_r[...])
        z_r[...] = z
        u = jnp.dot(z, w1_r[...], preferred_element_type=jnp.float32) + b1_r[...]
        e = jnp.where(u > 0, u, jnp.exp(jnp.minimum(u, 0.0)) - 1.0)
        p = jnp.dot(e, w2_r[...], preferred_element_type=jnp.float32) + b2_r[...]
        nrm = jnp.sqrt(jnp.sum(p * p, axis=1, keepdims=True))
        n_r[...] = p / (nrm + 1e-8)

    return pl.pallas_call(
        body,
        grid=(N // T,),
        in_specs=[
            pl.BlockSpec((T, N_OUT), lambda i: (i, 0)),
            pl.BlockSpec((T, 128), lambda i: (i, 0)),
            pl.BlockSpec((N_OUT, N_OUT), lambda i: (0, 0)),
            pl.BlockSpec((1, N_OUT), lambda i: (0, 0)),
            pl.BlockSpec((N_OUT, N_OUT), lambda i: (0, 0)),
            pl.BlockSpec((1, N_OUT), lambda i: (0, 0)),
        ],
        out_specs=[pl.BlockSpec((T, N_OUT), lambda i: (i, 0))] * 2,
        out_shape=[jax.ShapeDtypeStruct((N, N_OUT), jnp.float32)] * 2,
    )(q, deg_in, Wp1, bp1.reshape(1, N_OUT), Wp2, bp2.reshape(1, N_OUT))


def _contrast_half(na, nb, pos):
    T = 200

    def body(na_r, nb_r, pos_r, out_r, acc):
        i = pl.program_id(0)
        sim = jnp.exp(
            lax.dot_general(
                na_r[...], nb_r[...], (((1,), (1,)), ((), ())),
                preferred_element_type=jnp.float32,
            )
            * (1.0 / TAU)
        )
        rs = jnp.sum(sim, axis=1)
        sp = jnp.sum(sim * pos_r[...], axis=1)
        term = jnp.log(sp / (rs + 1e-8) + 1e-8)
        ssum = jnp.sum(term)

        @pl.when(i == 0)
        def _():
            acc[0] = 0.0

        acc[0] += ssum

        @pl.when(i == pl.num_programs(0) - 1)
        def _():
            out_r[...] = jnp.full((1, 128), acc[0], jnp.float32)

    return pl.pallas_call(
        body,
        grid=(N // T,),
        in_specs=[
            pl.BlockSpec((T, N_OUT), lambda i: (i, 0)),
            pl.BlockSpec((N, N_OUT), lambda i: (0, 0)),
            pl.BlockSpec((T, N), lambda i: (i, 0)),
        ],
        out_specs=pl.BlockSpec((1, 128), lambda i: (0, 0)),
        out_shape=jax.ShapeDtypeStruct((1, 128), jnp.float32),
        scratch_shapes=[pltpu.SMEM((1,), jnp.float32)],
    )(na, nb, pos)


def kernel(pos, g_pmi, g_emb, node_features, W1_pmi, W2_pmi, W1_emb, W2_emb, Wp1, bp1, Wp2, bp2):
    deg = _sc_degrees(g_pmi, g_emb)
    dop, dip = deg[0, 0], deg[0, 1]
    doe, die = deg[1, 0], deg[1, 1]

    xs_p, xs_e = _prep(node_features, dop, doe)
    a0 = _sc_scatter(xs_p[:, :128], xs_e[:, :128], g_pmi, g_emb)
    a1 = _sc_scatter(xs_p[:, 128:], xs_e[:, 128:], g_pmi, g_emb)

    y_p = _layer(a0[0], a1[0], dip, dop, W1_pmi, W2_pmi)
    y_e = _layer(a0[1], a1[1], die, doe, W1_emb, W2_emb)
    q = _sc_scatter(y_p, y_e, g_pmi, g_emb)

    z_p, n1 = _zproj(q[0], dip, Wp1, bp1, Wp2, bp2)
    z_e, n2 = _zproj(q[1], die, Wp1, bp1, Wp2, bp2)

    lA = _contrast_half(n1, n2, pos)[0, 0]
    lB = _contrast_half(n2, n1, pos)[0, 0]
    loss = LAM * (-lA / N) + (1.0 - LAM) * (-lB / N)
    return (z_p, z_e, loss)


# speedup vs baseline: 7.0047x; 1.0711x over previous
import functools

import jax
import jax.numpy as jnp
from jax import lax
from jax.experimental import pallas as pl
from jax.experimental.pallas import tpu as pltpu
from jax.experimental.pallas import tpu_sc as plsc

N = 10000
E = 160000
N_INP = 256
N_HID = 512
N_OUT = 128
TAU = 0.8
LAM = 0.5

_B = 128
_NBATCH = E // _B
_NSUB = 16
_MAXB = 80
_HC = 40
_GPAD = _NSUB * _MAXB
_WB = 632
_WB_LAST = N - (_NSUB - 1) * _WB


def _edges_2d(g):
    return jnp.pad(g.reshape(2, _NBATCH, _B), ((0, 0), (0, _GPAD - _NBATCH), (0, 0)))


def _sc_mesh():
    return plsc.VectorSubcoreMesh(
        core_axis_name="c", subcore_axis_name="s", num_cores=2, num_subcores=_NSUB
    )


def _tile_batches(s):
    start = pl.multiple_of(s * _MAXB, 8)
    nb = jnp.minimum(_MAXB, _NBATCH - s * _MAXB)
    return start, nb


def _writeback(s, copy_one):
    r0 = pl.multiple_of(s * _WB, 8)

    @pl.when(s < _NSUB - 1)
    def _():
        copy_one(r0, _WB)

    @pl.when(s == _NSUB - 1)
    def _():
        copy_one((_NSUB - 1) * _WB, _WB_LAST)


def _sc_degrees(g_pmi, g_emb):
    ones = jnp.ones((_B, 128), jnp.float32)
    zdeg = jnp.zeros((N, 128), jnp.float32)

    @functools.partial(
        pl.kernel,
        out_type=jax.ShapeDtypeStruct((2, 2, N, 128), jnp.float32),
        mesh=_sc_mesh(),
        scratch_types=[
            pltpu.VMEM((_MAXB, _B), jnp.int32),
            pltpu.VMEM((_B, 128), jnp.float32),
            pltpu.VMEM_SHARED((N, 128), jnp.float32),
        ],
    )
    def k(gp_h, ge_h, z_h, ones_h, out_h, sidx, ones_v, acc):
        c = lax.axis_index("c")
        s = lax.axis_index("s")
        pltpu.sync_copy(ones_h, ones_v)
        start, nb = _tile_batches(s)

        for kind in (0, 1):
            @pl.when(c == 0)
            def _():
                pltpu.sync_copy(gp_h.at[kind, pl.ds(start, _MAXB)], sidx)

            @pl.when(c == 1)
            def _():
                pltpu.sync_copy(ge_h.at[kind, pl.ds(start, _MAXB)], sidx)

            @pl.when(s == 0)
            def _():
                pltpu.sync_copy(z_h, acc)

            plsc.subcore_barrier()

            def body(i, carry):
                pltpu.sync_copy(ones_v, acc.at[sidx.at[i]], add=True)
                return carry

            lax.fori_loop(0, nb, body, 0)

            plsc.subcore_barrier()

            def copy_one(r0, nr):
                pltpu.sync_copy(acc.at[pl.ds(r0, nr)], out_h.at[c, kind, pl.ds(r0, nr)])

            _writeback(s, copy_one)
            plsc.subcore_barrier()

    return k(_edges_2d(g_pmi), _edges_2d(g_emb), zdeg, ones)


def _sc_scatter(v_pmi, v_emb, g_pmi, g_emb):
    z128 = jnp.zeros((N, 128), jnp.float32)

    @functools.partial(
        pl.kernel,
        out_type=jax.ShapeDtypeStruct((2, N, 128), jnp.float32),
        mesh=_sc_mesh(),
        scratch_types=[
            pltpu.VMEM((_HC, _B), jnp.int32),
            pltpu.VMEM((_HC, _B), jnp.int32),
            pltpu.VMEM((_B, 128), jnp.float32),
            pltpu.VMEM((_B, 128), jnp.float32),
            pltpu.VMEM_SHARED((N, 128), jnp.float32),
            pltpu.SemaphoreType.DMA,
            pltpu.SemaphoreType.DMA,
        ],
    )
    def k(vp_h, ve_h, gp_h, ge_h, z_h, out_h, sidx, didx, rows0, rows1, acc, sem0, sem1):
        c = lax.axis_index("c")
        s = lax.axis_index("s")
        start, nb = _tile_batches(s)

        @pl.when(s == 0)
        def _():
            pltpu.sync_copy(z_h, acc)

        plsc.subcore_barrier()

        def run(v_h, g_h):
            for h in range(_MAXB // _HC):
                h0 = pl.multiple_of(start + h * _HC, 8)
                nh = jnp.clip(nb - h * _HC, 0, _HC)

                @pl.when(nh > 0)
                def _():
                    pltpu.sync_copy(g_h.at[0, pl.ds(h0, _HC)], sidx)
                    pltpu.sync_copy(g_h.at[1, pl.ds(h0, _HC)], didx)
                    pltpu.async_copy(v_h.at[sidx.at[0]], rows0, sem0)

                    def body2(j, carry):
                        i0 = 2 * j
                        i1 = i0 + 1
                        pltpu.async_copy(v_h.at[sidx.at[i1]], rows1, sem1)
                        pltpu.make_async_copy(v_h.at[sidx.at[i0]], rows0, sem0).wait()
                        pltpu.sync_copy(rows0, acc.at[didx.at[i0]], add=True)

                        @pl.when(i0 + 2 < nh)
                        def _():
                            pltpu.async_copy(v_h.at[sidx.at[i0 + 2]], rows0, sem0)

                        pltpu.make_async_copy(v_h.at[sidx.at[i1]], rows1, sem1).wait()
                        pltpu.sync_copy(rows1, acc.at[didx.at[i1]], add=True)
                        return carry

                    lax.fori_loop(0, nh // 2, body2, 0)

        @pl.when(c == 0)
        def _():
            run(vp_h, gp_h)

        @pl.when(c == 1)
        def _():
            run(ve_h, ge_h)

        plsc.subcore_barrier()

        def copy_one(r0, nr):
            pltpu.sync_copy(acc.at[pl.ds(r0, nr)], out_h.at[c, pl.ds(r0, nr)])

        _writeback(s, copy_one)

    return k(v_pmi, v_emb, _edges_2d(g_pmi), _edges_2d(g_emb), z128)


def _inv_sqrt_deg(d16):
    d0 = d16[:, 0:1]
    return jnp.where(d0 > 0, lax.rsqrt(jnp.where(d0 > 0, d0, 1.0)), 0.0)


def _prep(x, deg_out_pmi, deg_out_emb):
    T = 2000

    def body(x_r, dp_r, de_r, xp_r, xe_r):
        xv = x_r[...]
        xp_r[...] = xv * _inv_sqrt_deg(dp_r[...])
        xe_r[...] = xv * _inv_sqrt_deg(de_r[...])

    return pl.pallas_call(
        body,
        grid=(N // T,),
        in_specs=[
            pl.BlockSpec((T, N_INP), lambda i: (i, 0)),
            pl.BlockSpec((T, 128), lambda i: (i, 0)),
            pl.BlockSpec((T, 128), lambda i: (i, 0)),
        ],
        out_specs=[pl.BlockSpec((T, N_INP), lambda i: (i, 0))] * 2,
        out_shape=[jax.ShapeDtypeStruct((N, N_INP), jnp.float32)] * 2,
    )(x, deg_out_pmi, deg_out_emb)


def _layer(a0, a1, deg_in, deg_out, W1, W2):
    T = 2000

    def body(a0_r, a1_r, di_r, do_r, w1_r, w2_r, y_r):
        ndst = _inv_sqrt_deg(di_r[...])
        nsrc = _inv_sqrt_deg(do_r[...])
        w1 = w1_r[...]
        h = jnp.dot(a0_r[...] * ndst, w1[0:128, :], preferred_element_type=jnp.float32)
        h += jnp.dot(a1_r[...] * ndst, w1[128:256, :], preferred_element_type=jnp.float32)
        h = jnp.maximum(h, 0.0) * nsrc
        y_r[...] = jnp.dot(h, w2_r[...], preferred_element_type=jnp.float32)

    return pl.pallas_call(
        body,
        grid=(N // T,),
        in_specs=[
            pl.BlockSpec((T, 128), lambda i: (i, 0)),
            pl.BlockSpec((T, 128), lambda i: (i, 0)),
            pl.BlockSpec((T, 128), lambda i: (i, 0)),
            pl.BlockSpec((T, 128), lambda i: (i, 0)),
            pl.BlockSpec((N_INP, N_HID), lambda i: (0, 0)),
            pl.BlockSpec((N_HID, N_OUT), lambda i: (0, 0)),
        ],
        out_specs=pl.BlockSpec((T, N_OUT), lambda i: (i, 0)),
        out_shape=jax.ShapeDtypeStruct((N, N_OUT), jnp.float32),
    )(a0, a1, deg_in, deg_out, W1, W2)


def _zproj(q, deg_in, Wp1, bp1, Wp2, bp2):
    T = 2000

    def body(q_r, di_r, w1_r, b1_r, w2_r, b2_r, z_r, n_r):
        z = q_r[...] * _inv_sqrt_deg(di_r[...])
        z_r[...] = z
        u = jnp.dot(z, w1_r[...], preferred_element_type=jnp.float32) + b1_r[...]
        e = jnp.where(u > 0, u, jnp.exp(jnp.minimum(u, 0.0)) - 1.0)
        p = jnp.dot(e, w2_r[...], preferred_element_type=jnp.float32) + b2_r[...]
        nrm = jnp.sqrt(jnp.sum(p * p, axis=1, keepdims=True))
        n_r[...] = p / (nrm + 1e-8)

    return pl.pallas_call(
        body,
        grid=(N // T,),
        in_specs=[
            pl.BlockSpec((T, N_OUT), lambda i: (i, 0)),
            pl.BlockSpec((T, 128), lambda i: (i, 0)),
            pl.BlockSpec((N_OUT, N_OUT), lambda i: (0, 0)),
            pl.BlockSpec((1, N_OUT), lambda i: (0, 0)),
            pl.BlockSpec((N_OUT, N_OUT), lambda i: (0, 0)),
            pl.BlockSpec((1, N_OUT), lambda i: (0, 0)),
        ],
        out_specs=[pl.BlockSpec((T, N_OUT), lambda i: (i, 0))] * 2,
        out_shape=[jax.ShapeDtypeStruct((N, N_OUT), jnp.float32)] * 2,
    )(q, deg_in, Wp1, bp1.reshape(1, N_OUT), Wp2, bp2.reshape(1, N_OUT))


def _contrast(n1, n2, pos):
    T = 200

    def body(n1_r, n2_r, n1f_r, n2f_r, pos_r, out_r, acc):
        i = pl.program_id(0)
        posv = pos_r[...]

        def half(a_strip, b_full):
            sim = jnp.exp(
                lax.dot_general(
                    a_strip, b_full, (((1,), (1,)), ((), ())),
                    preferred_element_type=jnp.float32,
                )
                * (1.0 / TAU)
            )
            rs = jnp.sum(sim, axis=1)
            sp = jnp.sum(sim * posv, axis=1)
            return jnp.sum(jnp.log(sp / (rs + 1e-8) + 1e-8))

        sA = half(n1_r[...], n2f_r[...])
        sB = half(n2_r[...], n1f_r[...])

        @pl.when(i == 0)
        def _():
            acc[0] = 0.0
            acc[1] = 0.0

        acc[0] += sA
        acc[1] += sB

        @pl.when(i == pl.num_programs(0) - 1)
        def _():
            out_r[...] = jnp.concatenate(
                [jnp.full((1, 128), acc[0], jnp.float32),
                 jnp.full((1, 128), acc[1], jnp.float32)], axis=0
            )

    return pl.pallas_call(
        body,
        grid=(N // T,),
        in_specs=[
            pl.BlockSpec((T, N_OUT), lambda i: (i, 0)),
            pl.BlockSpec((T, N_OUT), lambda i: (i, 0)),
            pl.BlockSpec((N, N_OUT), lambda i: (0, 0)),
            pl.BlockSpec((N, N_OUT), lambda i: (0, 0)),
            pl.BlockSpec((T, N), lambda i: (i, 0)),
        ],
        out_specs=pl.BlockSpec((2, 128), lambda i: (0, 0)),
        out_shape=jax.ShapeDtypeStruct((2, 128), jnp.float32),
        scratch_shapes=[pltpu.SMEM((2,), jnp.float32)],
    )(n1, n2, n1, n2, pos)


def kernel(pos, g_pmi, g_emb, node_features, W1_pmi, W2_pmi, W1_emb, W2_emb, Wp1, bp1, Wp2, bp2):
    deg = _sc_degrees(g_pmi, g_emb)
    dop, dip = deg[0, 0], deg[0, 1]
    doe, die = deg[1, 0], deg[1, 1]

    xs_p, xs_e = _prep(node_features, dop, doe)
    a0 = _sc_scatter(xs_p[:, :128], xs_e[:, :128], g_pmi, g_emb)
    a1 = _sc_scatter(xs_p[:, 128:], xs_e[:, 128:], g_pmi, g_emb)

    y_p = _layer(a0[0], a1[0], dip, dop, W1_pmi, W2_pmi)
    y_e = _layer(a0[1], a1[1], die, doe, W1_emb, W2_emb)
    q = _sc_scatter(y_p, y_e, g_pmi, g_emb)

    z_p, n1 = _zproj(q[0], dip, Wp1, bp1, Wp2, bp2)
    z_e, n2 = _zproj(q[1], die, Wp1, bp1, Wp2, bp2)

    l_sums = _contrast(n1, n2, pos)
    loss = LAM * (-l_sums[0, 0] / N) + (1.0 - LAM) * (-l_sums[1, 0] / N)
    return (z_p, z_e, loss)
